# baseline (speedup 1.0000x reference)
import functools

import jax
import jax.numpy as jnp
from jax import lax
from jax.experimental import pallas as pl
from jax.experimental.pallas import tpu as pltpu
from jax.experimental.pallas import tpu_sc as plsc

EMBED = 2048
NC = 2
NS = 16
NW = NC * NS
CHUNK = 16
NBUF = 3


def _emb_body(n_chunks, idx_hbm, table_hbm, out_hbm, idx_v, *scratch):
  bufs = scratch[:NBUF]
  sem_g = scratch[NBUF:2 * NBUF]
  sem_o = scratch[2 * NBUF:3 * NBUF]

  cid = lax.axis_index("c")
  sid = lax.axis_index("s")
  wid = sid * NC + cid
  base = wid * (n_chunks * CHUNK)

  pltpu.sync_copy(idx_hbm.at[wid], idx_v)

  for b in range(NBUF):
    pltpu.async_copy(table_hbm.at[idx_v.at[b]], bufs[b], sem_g[b])

  n_main = (n_chunks // NBUF) * NBUF
  if n_main == n_chunks:
    n_main -= NBUF

  @pl.loop(0, n_main, step=NBUF)
  def _(g):
    for b in range(NBUF):
      i = g + b
      pltpu.make_async_copy(table_hbm.at[idx_v.at[i]], bufs[b], sem_g[b]).wait()
      pltpu.async_copy(bufs[b], out_hbm.at[pl.ds(base + i * CHUNK, CHUNK)],
                       sem_o[b])
    for b in range(NBUF):
      i = g + b
      pltpu.make_async_copy(
          bufs[b], out_hbm.at[pl.ds(base + i * CHUNK, CHUNK)], sem_o[b]).wait()

      @pl.when(i + NBUF < n_chunks)
      def _():
        pltpu.async_copy(table_hbm.at[idx_v.at[i + NBUF]], bufs[b], sem_g[b])

  for i in range(n_main, n_chunks):
    b = i % NBUF
    pltpu.make_async_copy(table_hbm.at[idx_v.at[i]], bufs[b], sem_g[b]).wait()
    pltpu.async_copy(bufs[b], out_hbm.at[pl.ds(base + i * CHUNK, CHUNK)],
                     sem_o[b])
  for i in range(n_main, n_chunks):
    b = i % NBUF
    pltpu.make_async_copy(
        bufs[b], out_hbm.at[pl.ds(base + i * CHUNK, CHUNK)], sem_o[b]).wait()


def kernel(input_ids, embed_table):
  batch, seq = input_ids.shape
  total = batch * seq
  n_chunks = total // (NW * CHUNK)
  idx = input_ids.reshape(NW, n_chunks, CHUNK).astype(jnp.int32)

  mesh = plsc.VectorSubcoreMesh(core_axis_name="c", subcore_axis_name="s")
  k = pl.kernel(
      functools.partial(_emb_body, n_chunks),
      out_type=jax.ShapeDtypeStruct((total, EMBED), jnp.float32),
      mesh=mesh,
      scratch_types=(
          [pltpu.VMEM((n_chunks, CHUNK), jnp.int32)]
          + [pltpu.VMEM((CHUNK, EMBED), jnp.float32) for _ in range(NBUF)]
          + [pltpu.SemaphoreType.DMA for _ in range(2 * NBUF)]
      ),
  )
  out = k(idx, embed_table)
  return out.reshape(batch, seq, EMBED)


# speedup vs baseline: 1.0594x; 1.0594x over previous
import functools

import jax
import jax.numpy as jnp
from jax import lax
from jax.experimental import pallas as pl
from jax.experimental.pallas import tpu as pltpu
from jax.experimental.pallas import tpu_sc as plsc

EMBED = 2048
NC = 2
NS = 16
NW = NC * NS
CHUNK = 8
NBUF = 4


def _emb_body(n_chunks, idx_hbm, table_hbm, out_hbm, idx_v, *scratch):
  bufs = scratch[:NBUF]
  sem_g = scratch[NBUF:2 * NBUF]
  sem_o = scratch[2 * NBUF:3 * NBUF]

  cid = lax.axis_index("c")
  sid = lax.axis_index("s")
  wid = sid * NC + cid
  base = wid * (n_chunks * CHUNK)

  pltpu.sync_copy(idx_hbm.at[wid], idx_v)

  for b in range(NBUF):
    pltpu.async_copy(table_hbm.at[idx_v.at[b]], bufs[b], sem_g[b])

  @pl.loop(0, n_chunks, step=NBUF)
  def _(g):
    for b in range(NBUF):
      i = g + b
      pltpu.make_async_copy(table_hbm.at[idx_v.at[i]], bufs[b], sem_g[b]).wait()
      cp_out = pltpu.async_copy(
          bufs[b], out_hbm.at[pl.ds(base + i * CHUNK, CHUNK)], sem_o[b])
      cp_out.wait()

      @pl.when(i + NBUF < n_chunks)
      def _():
        pltpu.async_copy(table_hbm.at[idx_v.at[i + NBUF]], bufs[b], sem_g[b])


def kernel(input_ids, embed_table):
  batch, seq = input_ids.shape
  total = batch * seq
  n_chunks = total // (NW * CHUNK)
  idx = input_ids.reshape(NW, n_chunks, CHUNK).astype(jnp.int32)

  mesh = plsc.VectorSubcoreMesh(core_axis_name="c", subcore_axis_name="s")
  k = pl.kernel(
      functools.partial(_emb_body, n_chunks),
      out_type=jax.ShapeDtypeStruct((total, EMBED), jnp.float32),
      mesh=mesh,
      scratch_types=(
          [pltpu.VMEM((n_chunks, CHUNK), jnp.int32)]
          + [pltpu.VMEM((CHUNK, EMBED), jnp.float32) for _ in range(NBUF)]
          + [pltpu.SemaphoreType.DMA for _ in range(2 * NBUF)]
      ),
  )
  out = k(idx, embed_table)
  return out.reshape(batch, seq, EMBED)


# speedup vs baseline: 1.0607x; 1.0012x over previous
import functools

import jax
import jax.numpy as jnp
from jax import lax
from jax.experimental import pallas as pl
from jax.experimental.pallas import tpu as pltpu
from jax.experimental.pallas import tpu_sc as plsc

EMBED = 2048
NC = 2
NS = 16
NW = NC * NS
CHUNK = 16


def _emb_body(n_chunks, idx_hbm, table_hbm, out_hbm, idx_v, buf0, buf1,
              sem_g0, sem_g1, sem_o0, sem_o1):
  cid = lax.axis_index("c")
  sid = lax.axis_index("s")
  wid = sid * NC + cid
  base = wid * (n_chunks * CHUNK)

  pltpu.sync_copy(idx_hbm.at[wid], idx_v)

  pltpu.async_copy(table_hbm.at[idx_v.at[0]], buf0, sem_g0)
  pltpu.async_copy(table_hbm.at[idx_v.at[1]], buf1, sem_g1)

  @pl.loop(0, n_chunks, step=2)
  def _(g):
    for b, (buf, sem_g, sem_o) in enumerate(
        ((buf0, sem_g0, sem_o0), (buf1, sem_g1, sem_o1))):
      i = g + b
      pltpu.make_async_copy(table_hbm.at[idx_v.at[i]], buf, sem_g).wait()
      cp_out = pltpu.async_copy(
          buf, out_hbm.at[pl.ds(base + i * CHUNK, CHUNK)], sem_o)

      cp_out.wait()

      @pl.when(i + 2 < n_chunks)
      def _():
        pltpu.async_copy(table_hbm.at[idx_v.at[i + 2]], buf, sem_g)


def kernel(input_ids, embed_table):
  batch, seq = input_ids.shape
  total = batch * seq
  n_chunks = total // (NW * CHUNK)
  idx = input_ids.reshape(NW, n_chunks, CHUNK).astype(jnp.int32)

  mesh = plsc.VectorSubcoreMesh(core_axis_name="c", subcore_axis_name="s")
  k = pl.kernel(
      functools.partial(_emb_body, n_chunks),
      out_type=jax.ShapeDtypeStruct((total, EMBED), jnp.float32),
      mesh=mesh,
      scratch_types=[
          pltpu.VMEM((n_chunks, CHUNK), jnp.int32),
          pltpu.VMEM((CHUNK, EMBED), jnp.float32),
          pltpu.VMEM((CHUNK, EMBED), jnp.float32),
          pltpu.SemaphoreType.DMA,
          pltpu.SemaphoreType.DMA,
          pltpu.SemaphoreType.DMA,
          pltpu.SemaphoreType.DMA,
      ],
  )
  out = k(idx, embed_table)
  return out.reshape(batch, seq, EMBED)


# speedup vs baseline: 1.0668x; 1.0058x over previous
import functools

import jax
import jax.numpy as jnp
from jax import lax
from jax.experimental import pallas as pl
from jax.experimental.pallas import tpu as pltpu
from jax.experimental.pallas import tpu_sc as plsc

EMBED = 2048
NC = 2
NS = 16
NW = NC * NS
CHUNK = 16


def _emb_body(n_chunks, w_per_b, idx_hbm, table_hbm, out_hbm, idx_v,
              buf0, buf1, sem_g0, sem_g1, sem_o0, sem_o1):
  n_ids = n_chunks * CHUNK
  cid = lax.axis_index("c")
  sid = lax.axis_index("s")
  wid = sid * NC + cid
  batch = wid // w_per_b
  seq0 = (wid % w_per_b) * n_ids

  pltpu.sync_copy(idx_hbm.at[batch, pl.ds(seq0, n_ids)], idx_v)

  def idx_chunk(i):
    return idx_v.at[pl.ds(i * CHUNK, CHUNK)]

  def out_chunk(i):
    return out_hbm.at[batch, pl.ds(seq0 + i * CHUNK, CHUNK)]

  pltpu.async_copy(table_hbm.at[idx_chunk(0)], buf0, sem_g0)
  pltpu.async_copy(table_hbm.at[idx_chunk(1)], buf1, sem_g1)

  @pl.loop(0, n_chunks, step=2)
  def _(g):
    for b, (buf, sem_g, sem_o) in enumerate(
        ((buf0, sem_g0, sem_o0), (buf1, sem_g1, sem_o1))):
      i = g + b
      pltpu.make_async_copy(table_hbm.at[idx_chunk(i)], buf, sem_g).wait()
      cp_out = pltpu.async_copy(buf, out_chunk(i), sem_o)

      cp_out.wait()

      @pl.when(i + 2 < n_chunks)
      def _():
        pltpu.async_copy(table_hbm.at[idx_chunk(i + 2)], buf, sem_g)


def kernel(input_ids, embed_table):
  batch, seq = input_ids.shape
  if input_ids.dtype != jnp.int32:
    input_ids = input_ids.astype(jnp.int32)
  total = batch * seq
  n_chunks = total // (NW * CHUNK)
  w_per_b = NW // batch

  mesh = plsc.VectorSubcoreMesh(core_axis_name="c", subcore_axis_name="s")
  k = pl.kernel(
      functools.partial(_emb_body, n_chunks, w_per_b),
      out_type=jax.ShapeDtypeStruct((batch, seq, EMBED), jnp.float32),
      mesh=mesh,
      scratch_types=[
          pltpu.VMEM((n_chunks * CHUNK,), jnp.int32),
          pltpu.VMEM((CHUNK, EMBED), jnp.float32),
          pltpu.VMEM((CHUNK, EMBED), jnp.float32),
          pltpu.SemaphoreType.DMA,
          pltpu.SemaphoreType.DMA,
          pltpu.SemaphoreType.DMA,
          pltpu.SemaphoreType.DMA,
      ],
  )
  return k(input_ids, embed_table)
